# baseline (speedup 1.0000x reference)
import jax
import jax.numpy as jnp
from jax import lax
from jax.experimental import pallas as pl
from jax.experimental.pallas import tpu as pltpu

S_CHUNK = 256
F_BLK = 4096


def _cumsum_kernel(x_ref, o_ref, carry_ref):
    s = pl.program_id(2)

    @pl.when(s == 0)
    def _init():
        carry_ref[...] = jnp.zeros_like(carry_ref)

    ch = x_ref.shape[1]
    row = lax.broadcasted_iota(jnp.int32, (ch, ch), 0)
    col = lax.broadcasted_iota(jnp.int32, (ch, ch), 1)
    tril = (row >= col).astype(jnp.bfloat16)
    local = jnp.dot(tril, x_ref[0].astype(jnp.bfloat16),
                    preferred_element_type=jnp.float32)
    out = local + carry_ref[...]
    o_ref[0] = out
    carry_ref[...] = out[-1:, :]


def _cumsum_pallas(x):
    B, S, F = x.shape
    grid = (B, F // F_BLK, S // S_CHUNK)
    return pl.pallas_call(
        _cumsum_kernel,
        grid=grid,
        in_specs=[pl.BlockSpec((1, S_CHUNK, F_BLK), lambda b, f, s: (b, s, f))],
        out_specs=pl.BlockSpec((1, S_CHUNK, F_BLK), lambda b, f, s: (b, s, f)),
        out_shape=jax.ShapeDtypeStruct(x.shape, x.dtype),
        scratch_shapes=[pltpu.VMEM((1, F_BLK), x.dtype)],
        compiler_params=pltpu.CompilerParams(
            dimension_semantics=("parallel", "parallel", "arbitrary"),
        ),
    )(x)


import functools
from jax.experimental.pallas import tpu_sc as plsc

_NC, _NS, _L = 2, 16, 16


_CH_ROWS = 512
_CW = 128
_G = _CW // _L


def _sc_cumsum(x):
    B, S, F = x.shape
    rows = B * S
    x2d = x.reshape(rows, F)
    nworkers = _NC * _NS
    mesh = plsc.VectorSubcoreMesh(core_axis_name="c", subcore_axis_name="s")
    n_chunks = rows // _CH_ROWS
    chunks_per_batch = S // _CH_ROWS

    @functools.partial(
        pl.kernel, mesh=mesh,
        out_type=jax.ShapeDtypeStruct((rows, F), jnp.float32),
        scratch_types=[pltpu.VMEM((_CH_ROWS, _CW), jnp.float32)],
    )
    def k(x_hbm, o_hbm, buf):
        wid = lax.axis_index("s") * _NC + lax.axis_index("c")
        c0 = wid * _CW

        def chunk_body(j, carry):
            keep = jnp.where((j % chunks_per_batch) == 0, 0.0, 1.0)
            carry = tuple(c * keep for c in carry)
            r0 = j * _CH_ROWS
            pltpu.sync_copy(x_hbm.at[pl.ds(r0, _CH_ROWS), pl.ds(c0, _CW)], buf)

            def row(r, carry):
                new = []
                for g in range(_G):
                    acc = carry[g] + buf[r, pl.ds(g * _L, _L)]
                    buf[r, pl.ds(g * _L, _L)] = acc
                    new.append(acc)
                return tuple(new)

            carry = lax.fori_loop(0, _CH_ROWS, row, carry)
            pltpu.sync_copy(buf, o_hbm.at[pl.ds(r0, _CH_ROWS), pl.ds(c0, _CW)])
            return carry

        zeros = tuple(jnp.zeros((_L,), jnp.float32) for _ in range(_G))
        lax.fori_loop(0, n_chunks, chunk_body, zeros)

    return k(x2d).reshape(B, S, F)


def kernel(x):
    return _sc_cumsum(x)


# speedup vs baseline: 1.0625x; 1.0625x over previous
import jax
import jax.numpy as jnp
from jax import lax
from jax.experimental import pallas as pl
from jax.experimental.pallas import tpu as pltpu

S_CHUNK = 256
F_BLK = 4096


def _cumsum_kernel(x_ref, o_ref, carry_ref):
    s = pl.program_id(2)

    @pl.when(s == 0)
    def _init():
        carry_ref[...] = jnp.zeros_like(carry_ref)

    ch = x_ref.shape[1]
    row = lax.broadcasted_iota(jnp.int32, (ch, ch), 0)
    col = lax.broadcasted_iota(jnp.int32, (ch, ch), 1)
    tril = (row >= col).astype(jnp.bfloat16)
    local = jnp.dot(tril, x_ref[0].astype(jnp.bfloat16),
                    preferred_element_type=jnp.float32)
    out = local + carry_ref[...]
    o_ref[0] = out
    carry_ref[...] = out[-1:, :]


def _cumsum_pallas(x):
    B, S, F = x.shape
    grid = (B, F // F_BLK, S // S_CHUNK)
    return pl.pallas_call(
        _cumsum_kernel,
        grid=grid,
        in_specs=[pl.BlockSpec((1, S_CHUNK, F_BLK), lambda b, f, s: (b, s, f))],
        out_specs=pl.BlockSpec((1, S_CHUNK, F_BLK), lambda b, f, s: (b, s, f)),
        out_shape=jax.ShapeDtypeStruct(x.shape, x.dtype),
        scratch_shapes=[pltpu.VMEM((1, F_BLK), x.dtype)],
        compiler_params=pltpu.CompilerParams(
            dimension_semantics=("parallel", "parallel", "arbitrary"),
        ),
    )(x)


import functools
from jax.experimental.pallas import tpu_sc as plsc

_NC, _NS, _L = 2, 16, 16


_CH_ROWS = 256
_CW = 128
_G = _CW // _L


def _sc_cumsum(x):
    B, S, F = x.shape
    rows = B * S
    x2d = x.reshape(rows, F)
    mesh = plsc.VectorSubcoreMesh(core_axis_name="c", subcore_axis_name="s")
    n_chunks = rows // _CH_ROWS
    chunks_per_batch = S // _CH_ROWS
    n_outer = n_chunks // 2

    @functools.partial(
        pl.kernel, mesh=mesh,
        out_type=jax.ShapeDtypeStruct((rows, F), jnp.float32),
        scratch_types=[
            pltpu.VMEM((_CH_ROWS, _CW), jnp.float32),
            pltpu.VMEM((_CH_ROWS, _CW), jnp.float32),
            pltpu.SemaphoreType.DMA,
            pltpu.SemaphoreType.DMA,
        ],
    )
    def k(x_hbm, o_hbm, buf0, buf1, so0, so1):
        wid = lax.axis_index("s") * _NC + lax.axis_index("c")
        c0 = wid * _CW
        bufs = (buf0, buf1)
        souts = (so0, so1)

        def src(j):
            return x_hbm.at[pl.ds(j * _CH_ROWS, _CH_ROWS), pl.ds(c0, _CW)]

        def dst(j):
            return o_hbm.at[pl.ds(j * _CH_ROWS, _CH_ROWS), pl.ds(c0, _CW)]

        def compute(buf, carry):
            def row(r, carry):
                new = []
                for g in range(_G):
                    acc = carry[g] + buf[r, pl.ds(g * _L, _L)]
                    buf[r, pl.ds(g * _L, _L)] = acc
                    new.append(acc)
                return tuple(new)

            return lax.fori_loop(0, _CH_ROWS, row, carry)

        def outer(j2, carry):
            for b in range(2):
                j = j2 * 2 + b

                @pl.when(j2 > 0)
                def _drain():
                    pltpu.make_async_copy(bufs[b], dst(j - 2), souts[b]).wait()

                pltpu.sync_copy(src(j), bufs[b])
                keep = jnp.where((j % chunks_per_batch) == 0, 0.0, 1.0)
                carry = tuple(c * keep for c in carry)
                carry = compute(bufs[b], carry)
                pltpu.async_copy(bufs[b], dst(j), souts[b])
            return carry

        zeros = tuple(jnp.zeros((_L,), jnp.float32) for _ in range(_G))
        lax.fori_loop(0, n_outer, outer, zeros)
        pltpu.make_async_copy(buf0, dst(n_chunks - 2), so0).wait()
        pltpu.make_async_copy(buf1, dst(n_chunks - 1), so1).wait()

    return k(x2d).reshape(B, S, F)


def kernel(x):
    return _sc_cumsum(x)


# speedup vs baseline: 1.8068x; 1.7005x over previous
import jax
import jax.numpy as jnp
from jax import lax
from jax.experimental import pallas as pl
from jax.experimental.pallas import tpu as pltpu

S_CHUNK = 256
F_BLK = 4096


def _cumsum_kernel(x_ref, o_ref, carry_ref):
    s = pl.program_id(2)

    @pl.when(s == 0)
    def _init():
        carry_ref[...] = jnp.zeros_like(carry_ref)

    ch = x_ref.shape[1]
    row = lax.broadcasted_iota(jnp.int32, (ch, ch), 0)
    col = lax.broadcasted_iota(jnp.int32, (ch, ch), 1)
    tril = (row >= col).astype(jnp.bfloat16)
    local = jnp.dot(tril, x_ref[0].astype(jnp.bfloat16),
                    preferred_element_type=jnp.float32)
    out = local + carry_ref[...]
    o_ref[0] = out
    carry_ref[...] = out[-1:, :]


def kernel(x):
    B, S, F = x.shape
    grid = (B, F // F_BLK, S // S_CHUNK)
    return pl.pallas_call(
        _cumsum_kernel,
        grid=grid,
        in_specs=[pl.BlockSpec((1, S_CHUNK, F_BLK), lambda b, f, s: (b, s, f))],
        out_specs=pl.BlockSpec((1, S_CHUNK, F_BLK), lambda b, f, s: (b, s, f)),
        out_shape=jax.ShapeDtypeStruct(x.shape, x.dtype),
        scratch_shapes=[pltpu.VMEM((1, F_BLK), x.dtype)],
        compiler_params=pltpu.CompilerParams(
            dimension_semantics=("parallel", "parallel", "arbitrary"),
        ),
    )(x)
